# baseline (speedup 1.0000x reference)
import functools

import jax
import jax.numpy as jnp
from jax import lax
from jax.experimental import pallas as pl
from jax.experimental.pallas import tpu as pltpu
from jax.experimental.pallas import tpu_sc as plsc


def _sc_gather(table, idx, num_workers=32):
    n = idx.shape[0]
    d = table.shape[1]
    b_per_w = n // num_workers
    mesh = plsc.VectorSubcoreMesh(core_axis_name="c", subcore_axis_name="s")

    @functools.partial(
        pl.kernel,
        mesh=mesh,
        out_type=jax.ShapeDtypeStruct((n, d), table.dtype),
        scratch_types=[
            pltpu.VMEM((b_per_w,), jnp.int32),
            pltpu.VMEM((b_per_w, d), table.dtype),
            pltpu.SemaphoreType.DMA,
        ],
        compiler_params=pltpu.CompilerParams(use_tc_tiling_on_sc=False),
    )
    def gather_kernel(table_hbm, idx_hbm, out_hbm, idx_v, rows_v, sem):
        wid = lax.axis_index("s") * 2 + lax.axis_index("c")
        base = wid * b_per_w
        pltpu.sync_copy(idx_hbm.at[pl.ds(base, b_per_w)], idx_v)
        pltpu.async_copy(table_hbm.at[idx_v], rows_v, sem).wait()
        pltpu.sync_copy(rows_v, out_hbm.at[pl.ds(base, b_per_w)])

    return gather_kernel(table, idx)


def _mm_body(e_ref, w_ref, b_ref, o_ref):
    o_ref[...] = (
        lax.dot_general(
            e_ref[...],
            w_ref[...],
            (((1,), (1,)), ((), ())),
            preferred_element_type=jnp.float32,
        )
        + b_ref[...]
    )


def _tc_matmul(embeds, W, b, tile_v=2048):
    B, K = embeds.shape
    V = W.shape[0]
    grid = pl.cdiv(V, tile_v)
    return pl.pallas_call(
        _mm_body,
        grid=(grid,),
        in_specs=[
            pl.BlockSpec((B, K), lambda i: (0, 0)),
            pl.BlockSpec((tile_v, K), lambda i: (i, 0)),
            pl.BlockSpec((1, tile_v), lambda i: (0, i)),
        ],
        out_specs=pl.BlockSpec((B, tile_v), lambda i: (0, i)),
        out_shape=jax.ShapeDtypeStruct((B, V), jnp.float32),
    )(embeds, W, b.reshape(1, V))


def kernel(x, emb, W, b):
    B, ctx = x.shape
    d = emb.shape[1]
    idx = x.reshape(-1).astype(jnp.int32)
    rows = _sc_gather(emb, idx)
    embeds = rows.reshape(B, ctx * d)
    return _tc_matmul(embeds, W, b)
